# baseline (speedup 1.0000x reference)
import functools

import jax
import jax.numpy as jnp
from jax import lax
from jax.experimental import pallas as pl
from jax.experimental.pallas import tpu as pltpu
from jax.experimental.pallas import tpu_sc as plsc

_B, _N, _D = 16, 512, 256
_NC = 2
_NS = 16
_LANES = 16
_NW = _NC * _NS
_ROWS_PER_TILE = _N // _NW
_TILE_ELEMS = _ROWS_PER_TILE * _N
_GROUPS_PER_ROW = _N // _LANES


_EI_COLS = _N * _N


def _edge_index_body(x_ref, ei_ref, xb_ref, pat_ref):
    c = pl.program_id(0)

    @pl.when(c == 0)
    def _():
        col = lax.broadcasted_iota(jnp.int32, (2, _EI_COLS), 1)
        row = lax.broadcasted_iota(jnp.int32, (2, _EI_COLS), 0)
        v0 = col >> 9
        v1 = col & (_N - 1)
        pat_ref[...] = jnp.where(row == 0, v0, v1)

    ei_ref[...] = pat_ref[...] + c * _N
    xb_ref[...] = x_ref[0]


_edge_index_tc = pl.pallas_call(
    _edge_index_body,
    grid=(_B,),
    in_specs=[pl.BlockSpec((1, _N, _D), lambda c: (c, 0, 0))],
    out_specs=[
        pl.BlockSpec((2, _EI_COLS), lambda c: (0, c)),
        pl.BlockSpec((_N, _D), lambda c: (c, 0)),
    ],
    out_shape=[
        jax.ShapeDtypeStruct((2, _B * _N * _N), jnp.int32),
        jax.ShapeDtypeStruct((_B * _N, _D), jnp.float32),
    ],
    scratch_shapes=[pltpu.VMEM((2, _EI_COLS), jnp.int32)],
)


def _edge_attr_body(adj_hbm, adjt_hbm, out_hbm, va, vb, sbuf, sem_in, sem_out):
    wid = lax.axis_index("s") * _NC + lax.axis_index("c")
    r0 = wid * _ROWS_PER_TILE
    cp_a = pltpu.make_async_copy(
        adj_hbm.at[pl.ds(r0, _ROWS_PER_TILE), :], va, sem_in)
    cp_b = pltpu.make_async_copy(
        adjt_hbm.at[pl.ds(r0, _ROWS_PER_TILE), :], vb, sem_in)
    cp_a.start()
    cp_b.start()
    cp_a.wait()
    cp_b.wait()

    @plsc.parallel_loop(0, _ROWS_PER_TILE * _GROUPS_PER_ROW, unroll=8)
    def _fill(g):
        i = g // _GROUPS_PER_ROW
        k = (g % _GROUPS_PER_ROW) * _LANES
        a = va[i, pl.ds(k, _LANES)]
        b = vb[i, pl.ds(k, _LANES)]
        sa = 1.0 / (1.0 + jnp.exp(-a))
        sb = 1.0 / (1.0 + jnp.exp(-b))
        sbuf[pl.ds(g * _LANES, _LANES)] = (sa + sb) * 0.5

    copies = []
    for b in range(_B):
        cp = pltpu.make_async_copy(
            sbuf,
            out_hbm.at[pl.ds(b * _N * _N + wid * _TILE_ELEMS, _TILE_ELEMS)],
            sem_out)
        cp.start()
        copies.append(cp)
    for cp in copies:
        cp.wait()


@functools.lru_cache(maxsize=1)
def _edge_attr_sc():
    return functools.partial(
        pl.kernel,
        out_type=jax.ShapeDtypeStruct((_B * _N * _N,), jnp.float32),
        mesh=plsc.VectorSubcoreMesh(core_axis_name="c", subcore_axis_name="s"),
        scratch_types=[
            pltpu.VMEM((_ROWS_PER_TILE, _N), jnp.float32),
            pltpu.VMEM((_ROWS_PER_TILE, _N), jnp.float32),
            pltpu.VMEM((_TILE_ELEMS,), jnp.float32),
            pltpu.SemaphoreType.DMA,
            pltpu.SemaphoreType.DMA,
        ],
    )(_edge_attr_body)


def kernel(x, adj):
    edge_index, x_batched = _edge_index_tc(x)
    edge_attr = _edge_attr_sc()(adj, adj.T)
    return x_batched, edge_index, edge_attr


# speedup vs baseline: 1.0152x; 1.0152x over previous
import functools

import jax
import jax.numpy as jnp
from jax import lax
from jax.experimental import pallas as pl
from jax.experimental.pallas import tpu as pltpu
from jax.experimental.pallas import tpu_sc as plsc

_B, _N, _D = 16, 512, 256
_NC = 2
_NS = 16
_LANES = 16
_NW = _NC * _NS
_ROWS_PER_TILE = _N // _NW
_TILE_ELEMS = _ROWS_PER_TILE * _N
_GROUPS_PER_ROW = _N // _LANES


_EI_COLS = _N * _N


def _edge_index_body(ei_ref, pat_ref):
    c = pl.program_id(0)

    @pl.when(c == 0)
    def _():
        col = lax.broadcasted_iota(jnp.int32, (2, _EI_COLS), 1)
        row = lax.broadcasted_iota(jnp.int32, (2, _EI_COLS), 0)
        v0 = col >> 9
        v1 = col & (_N - 1)
        pat_ref[...] = jnp.where(row == 0, v0, v1)

    ei_ref[...] = pat_ref[...] + c * _N


_edge_index_tc = pl.pallas_call(
    _edge_index_body,
    grid=(_B,),
    out_specs=pl.BlockSpec((2, _EI_COLS), lambda c: (0, c)),
    out_shape=jax.ShapeDtypeStruct((2, _B * _N * _N), jnp.int32),
    scratch_shapes=[pltpu.VMEM((2, _EI_COLS), jnp.int32)],
)


def _edge_attr_body(adj_hbm, adjt_hbm, out_hbm, va, vb, sbuf, sem_in, sem_out):
    wid = lax.axis_index("s") * _NC + lax.axis_index("c")
    r0 = wid * _ROWS_PER_TILE
    cp_a = pltpu.make_async_copy(
        adj_hbm.at[pl.ds(r0, _ROWS_PER_TILE), :], va, sem_in)
    cp_b = pltpu.make_async_copy(
        adjt_hbm.at[pl.ds(r0, _ROWS_PER_TILE), :], vb, sem_in)
    cp_a.start()
    cp_b.start()
    cp_a.wait()
    cp_b.wait()

    @plsc.parallel_loop(0, _ROWS_PER_TILE * _GROUPS_PER_ROW, unroll=8)
    def _fill(g):
        i = g // _GROUPS_PER_ROW
        k = (g % _GROUPS_PER_ROW) * _LANES
        a = va[i, pl.ds(k, _LANES)]
        b = vb[i, pl.ds(k, _LANES)]
        sa = 1.0 / (1.0 + jnp.exp(-a))
        sb = 1.0 / (1.0 + jnp.exp(-b))
        sbuf[pl.ds(g * _LANES, _LANES)] = (sa + sb) * 0.5

    copies = []
    for b in range(_B):
        cp = pltpu.make_async_copy(
            sbuf,
            out_hbm.at[pl.ds(b * _N * _N + wid * _TILE_ELEMS, _TILE_ELEMS)],
            sem_out)
        cp.start()
        copies.append(cp)
    for cp in copies:
        cp.wait()


@functools.lru_cache(maxsize=1)
def _edge_attr_sc():
    return functools.partial(
        pl.kernel,
        out_type=jax.ShapeDtypeStruct((_B * _N * _N,), jnp.float32),
        mesh=plsc.VectorSubcoreMesh(core_axis_name="c", subcore_axis_name="s"),
        scratch_types=[
            pltpu.VMEM((_ROWS_PER_TILE, _N), jnp.float32),
            pltpu.VMEM((_ROWS_PER_TILE, _N), jnp.float32),
            pltpu.VMEM((_TILE_ELEMS,), jnp.float32),
            pltpu.SemaphoreType.DMA,
            pltpu.SemaphoreType.DMA,
        ],
    )(_edge_attr_body)


def kernel(x, adj):
    Bv, Nv, Dv = x.shape
    x_batched = x.reshape(Bv * Nv, Dv)
    adjt = adj.T
    adjt, x_batched = lax.optimization_barrier((adjt, x_batched))
    edge_attr = _edge_attr_sc()(adj, adjt)
    edge_index = _edge_index_tc()
    return x_batched, edge_index, edge_attr
